# baseline (speedup 1.0000x reference)
import functools

import jax
import jax.numpy as jnp
from jax import lax
from jax.experimental import pallas as pl
from jax.experimental.pallas import tpu as pltpu
from jax.experimental.pallas import tpu_sc as plsc

N = 10000
F = 128
H = 128
C = 2
E = 320000
G = 64

NC = 2
NS = 16
L = 16
NW = NC * NS
NPN = 640
NPAD = NS * NPN
K = 80
EPW = E // NW
EPT = E // NS
CHB = EPW // K
CHC = EPT // K
GB = 5
NGB = CHB // GB
GA = 5
GC = 10
GC2 = GC // 2
NGC = CHC // GC2
R = 1024
GRID = NPAD // R

_mesh = plsc.VectorSubcoreMesh(
    core_axis_name="c", subcore_axis_name="s", num_cores=NC, num_subcores=NS)

_f32 = jnp.float32


def _zero_vec(ref, n):
    for j in range(n // L):
        ref[pl.ds(j * L, L)] = jnp.zeros((L,), _f32)


CA = E // NS // K
GA2 = 5
NGA = CA // GA2


@functools.partial(
    pl.kernel,
    out_type=jax.ShapeDtypeStruct((NPAD,), _f32),
    mesh=_mesh,
    scratch_types=[pltpu.VMEM((K,), jnp.int32)] * (2 * GA2) + [
        pltpu.VMEM((K,), _f32),
        pltpu.VMEM((K,), _f32),
        pltpu.VMEM((NPN,), _f32),
        pltpu.VMEM((NPN,), _f32),
        pltpu.VMEM_SHARED((NPAD,), _f32),
        pltpu.SemaphoreType.DMA,
        pltpu.SemaphoreType.DMA,
        pltpu.SemaphoreType.DMA,
        pltpu.SemaphoreType.DMA,
    ],
)
def _deg_kernel(dst_hbm, dinv_hbm, *rest):
    ibufs = rest[:2 * GA2]
    (ones_v, zbuf_v, degc_v, dinvc_v, deg_sp,
     isem0, isem1, ssem0, ssem1) = rest[2 * GA2:]
    bank_i = (ibufs[:GA2], ibufs[GA2:])
    isems = (isem0, isem1)
    ssems = (ssem0, ssem1)
    cid = lax.axis_index("c")
    sid = lax.axis_index("s")
    for j in range(K // L):
        ones_v[pl.ds(j * L, L)] = jnp.full((L,), 1.0, _f32)
    _zero_vec(zbuf_v, K)
    base_n = sid * NPN

    @pl.when(cid == 0)
    def _():
        def zr(j, carry):
            pltpu.sync_copy(zbuf_v, deg_sp.at[pl.ds(base_n + j * K, K)])
            return carry

        lax.fori_loop(0, NPN // K, zr, 0)

    plsc.subcore_barrier()

    @pl.when(cid == 0)
    def _():
        base_e = sid * (E // NS)

        def fire_i(i, b, bk):
            pltpu.async_copy(dst_hbm.at[pl.ds(base_e + i * K, K)],
                             bank_i[bk][b], isems[bk])

        def drain_i(i, b, bk):
            pltpu.make_async_copy(dst_hbm.at[pl.ds(base_e + i * K, K)],
                                  bank_i[bk][b], isems[bk]).wait()

        def fire_s(i, b, bk):
            return pltpu.async_copy(ones_v, deg_sp.at[bank_i[bk][b]],
                                    ssems[bk], add=True)

        for b in range(GA2):
            fire_i(b, b, 0)

        def pair(t, carry):
            a0 = (2 * t) * GA2
            a1 = (2 * t + 1) * GA2
            a2 = (2 * t + 2) * GA2
            for b in range(GA2):
                drain_i(a0 + b, b, 0)
            sd0 = [fire_s(a0 + b, b, 0) for b in range(GA2)]
            for b in range(GA2):
                fire_i(a1 + b, b, 1)
            for d in sd0:
                d.wait()
            for b in range(GA2):
                fire_i(a2 + b, b, 0)
            for b in range(GA2):
                drain_i(a1 + b, b, 1)
            sd1 = [fire_s(a1 + b, b, 1) for b in range(GA2)]
            for d in sd1:
                d.wait()
            return carry

        lax.fori_loop(0, NGA // 2 - 1, pair, 0)
        aP = (NGA - 2) * GA2
        aQ = (NGA - 1) * GA2
        for b in range(GA2):
            drain_i(aP + b, b, 0)
        sdP = [fire_s(aP + b, b, 0) for b in range(GA2)]
        for b in range(GA2):
            fire_i(aQ + b, b, 1)
        for d in sdP:
            d.wait()
        for b in range(GA2):
            drain_i(aQ + b, b, 1)
        sdQ = [fire_s(aQ + b, b, 1) for b in range(GA2)]
        for d in sdQ:
            d.wait()

    plsc.subcore_barrier()

    @pl.when(cid == 0)
    def _():
        pltpu.sync_copy(deg_sp.at[pl.ds(base_n, NPN)], degc_v)

        def nstep(j, carry):
            o = j * L
            d = degc_v[pl.ds(o, L)] + 1.0
            i32 = lax.bitcast_convert_type(d, jnp.int32)
            i32 = jnp.full((L,), 0x5F3759DF, jnp.int32) - (i32 >> 1)
            y = lax.bitcast_convert_type(i32, _f32)
            for _ in range(3):
                y = y * (1.5 - 0.5 * d * y * y)
            dinvc_v[pl.ds(o, L)] = y
            return carry

        lax.fori_loop(0, NPN // L, nstep, 0)
        pltpu.sync_copy(dinvc_v, dinv_hbm.at[pl.ds(base_n, NPN)])


KB = 40
CB2 = EPW // KB
GB = 3
NGB = 83


@functools.partial(
    pl.kernel,
    out_type=jax.ShapeDtypeStruct((NC, NPAD, H), _f32),
    mesh=_mesh,
    scratch_types=[
        pltpu.VMEM((EPW,), jnp.int32),
    ] + [pltpu.VMEM((KB, H), _f32)] * (2 * GB)
      + [pltpu.VMEM((KB,), jnp.int32)] * (2 * GB) + [
        pltpu.VMEM_SHARED((NPAD, H), _f32),
        pltpu.SemaphoreType.DMA,
        pltpu.SemaphoreType.DMA,
        pltpu.SemaphoreType.DMA,
        pltpu.SemaphoreType.DMA,
    ],
)
def _acc_kernel(hh_hbm, src_hbm, dst_hbm, accp_hbm, sidx_all, *rest):
    rows = rest[:2 * GB]
    ibufs = rest[2 * GB:4 * GB]
    acc_sp, gsem0, gsem1, ssem0, ssem1 = rest[4 * GB:]
    bank_r = (rows[:GB], rows[GB:])
    bank_i = (ibufs[:GB], ibufs[GB:])
    gsems = (gsem0, gsem1)
    ssems = (ssem0, ssem1)
    cid = lax.axis_index("c")
    sid = lax.axis_index("s")
    wid = sid * NC + cid
    zrow = rows[0]

    def zf(i, carry):
        r = i // (H // L)
        c8 = (i % (H // L)) * L
        zrow[r, pl.ds(c8, L)] = jnp.zeros((L,), _f32)
        return carry

    lax.fori_loop(0, KB * (H // L), zf, 0)
    base_n = sid * NPN

    def zr(j, carry):
        pltpu.sync_copy(zrow, acc_sp.at[pl.ds(base_n + j * KB, KB)])
        return carry

    lax.fori_loop(0, NPN // KB, zr, 0)
    base_e = wid * EPW
    pltpu.sync_copy(src_hbm.at[pl.ds(base_e, EPW)], sidx_all)
    plsc.subcore_barrier()

    def g_src(i):
        return hh_hbm.at[sidx_all.at[pl.ds(i * KB, KB)]]

    def i_src(i):
        return dst_hbm.at[pl.ds(base_e + i * KB, KB)]

    def fire_g(i, b, bk):
        pltpu.async_copy(g_src(i), bank_r[bk][b], gsems[bk])
        pltpu.async_copy(i_src(i), bank_i[bk][b], gsems[bk])

    def drain_g(i, b, bk):
        pltpu.make_async_copy(g_src(i), bank_r[bk][b], gsems[bk]).wait()
        pltpu.make_async_copy(i_src(i), bank_i[bk][b], gsems[bk]).wait()

    def fire_s(i, b, bk):
        return pltpu.async_copy(
            bank_r[bk][b], acc_sp.at[bank_i[bk][b]], ssems[bk], add=True)

    for b in range(GB):
        fire_g(b, b, 0)

    def pair(t, carry):
        a0 = (2 * t) * GB
        a1 = (2 * t + 1) * GB
        a2 = (2 * t + 2) * GB
        for b in range(GB):
            drain_g(a0 + b, b, 0)
        sd0 = [fire_s(a0 + b, b, 0) for b in range(GB)]
        for b in range(GB):
            fire_g(a1 + b, b, 1)
        for d in sd0:
            d.wait()
        for b in range(GB):
            fire_g(a2 + b, b, 0)
        for b in range(GB):
            drain_g(a1 + b, b, 1)
        sd1 = [fire_s(a1 + b, b, 1) for b in range(GB)]
        for d in sd1:
            d.wait()
        return carry

    lax.fori_loop(0, (NGB - 1) // 2, pair, 0)
    aL = (NGB - 1) * GB
    for b in range(GB):
        drain_g(aL + b, b, 0)
    sdL = [fire_s(aL + b, b, 0) for b in range(GB)]
    for d in sdL:
        d.wait()
    for i in range(NGB * GB, CB2):
        fire_g(i, 0, 0)
        drain_g(i, 0, 0)
        fire_s(i, 0, 0).wait()
    plsc.subcore_barrier()
    pltpu.sync_copy(acc_sp.at[pl.ds(base_n, NPN)],
                    accp_hbm.at[cid, pl.ds(base_n, NPN)])


CC = EPW // K
GC2 = 5
NGC = CC // GC2


@functools.partial(
    pl.kernel,
    out_type=(jax.ShapeDtypeStruct((NC, G), _f32),
              jax.ShapeDtypeStruct((NC, G), _f32)),
    mesh=_mesh,
    scratch_types=[
        pltpu.VMEM((EPW,), jnp.int32),
    ] + [pltpu.VMEM((K,), _f32)] * (4 * GC2)
      + [pltpu.VMEM((K,), jnp.int32)] * (2 * GC2) + [
        pltpu.VMEM((K,), _f32),
        pltpu.VMEM((NPN,), _f32),
        pltpu.VMEM((NPN,), _f32),
        pltpu.VMEM((NPN,), _f32),
        pltpu.VMEM((NPN,), _f32),
        pltpu.VMEM((NPN,), _f32),
        pltpu.VMEM((NPN,), _f32),
        pltpu.VMEM((NPN,), _f32),
        pltpu.VMEM((128,), jnp.int32),
        pltpu.VMEM_SHARED((NPAD,), _f32),
        pltpu.VMEM_SHARED((NPAD,), _f32),
        pltpu.VMEM_SHARED((G,), _f32),
        pltpu.VMEM_SHARED((G,), _f32),
        pltpu.SemaphoreType.DMA,
        pltpu.SemaphoreType.DMA,
        pltpu.SemaphoreType.DMA,
        pltpu.SemaphoreType.DMA,
    ],
)
def _pool_kernel(src_hbm, dst_hbm, zz0_hbm, zz1_hbm, zzb0_hbm, zzb1_hbm,
                 dinv_hbm, bidx_hbm, out0_hbm, out1_hbm,
                 sidx_all, *rest):
    vbufs = rest[:4 * GC2]
    ibufs = rest[4 * GC2:6 * GC2]
    (zbuf_v, t0c_v, t1c_v, zb0_v, zb1_v, dv_v, u0_v, u1_v, bibuf_v,
     t0_sp, t1_sp, pool0_sp, pool1_sp,
     gsem0, gsem1, ssem0, ssem1) = rest[6 * GC2:]
    bank_0 = (vbufs[:GC2], vbufs[GC2:2 * GC2])
    bank_1 = (vbufs[2 * GC2:3 * GC2], vbufs[3 * GC2:])
    bank_i = (ibufs[:GC2], ibufs[GC2:])
    gsems = (gsem0, gsem1)
    ssems = (ssem0, ssem1)
    cid = lax.axis_index("c")
    sid = lax.axis_index("s")
    wid = sid * NC + cid
    _zero_vec(zbuf_v, K)
    base_n = sid * NPN

    def zr(j, carry):
        pltpu.sync_copy(zbuf_v, t0_sp.at[pl.ds(base_n + j * K, K)])
        pltpu.sync_copy(zbuf_v, t1_sp.at[pl.ds(base_n + j * K, K)])
        return carry

    lax.fori_loop(0, NPN // K, zr, 0)
    base_e = wid * EPW
    pltpu.sync_copy(src_hbm.at[pl.ds(base_e, EPW)], sidx_all)

    @pl.when(sid == 0)
    def _():
        pltpu.sync_copy(zbuf_v.at[pl.ds(0, G)], pool0_sp)
        pltpu.sync_copy(zbuf_v.at[pl.ds(0, G)], pool1_sp)

    plsc.subcore_barrier()

    def fire_g(i, b, bk):
        pltpu.async_copy(dst_hbm.at[pl.ds(base_e + i * K, K)],
                         bank_i[bk][b], gsems[bk])
        pltpu.async_copy(zz0_hbm.at[sidx_all.at[pl.ds(i * K, K)]],
                         bank_0[bk][b], gsems[bk])
        pltpu.async_copy(zz1_hbm.at[sidx_all.at[pl.ds(i * K, K)]],
                         bank_1[bk][b], gsems[bk])

    def drain_g(i, b, bk):
        pltpu.make_async_copy(dst_hbm.at[pl.ds(base_e + i * K, K)],
                              bank_i[bk][b], gsems[bk]).wait()
        pltpu.make_async_copy(zz0_hbm.at[sidx_all.at[pl.ds(i * K, K)]],
                              bank_0[bk][b], gsems[bk]).wait()
        pltpu.make_async_copy(zz1_hbm.at[sidx_all.at[pl.ds(i * K, K)]],
                              bank_1[bk][b], gsems[bk]).wait()

    def fire_s(i, b, bk):
        return [pltpu.async_copy(bank_0[bk][b], t0_sp.at[bank_i[bk][b]],
                                 ssems[bk], add=True),
                pltpu.async_copy(bank_1[bk][b], t1_sp.at[bank_i[bk][b]],
                                 ssems[bk], add=True)]

    for b in range(GC2):
        fire_g(b, b, 0)

    def pair(t, carry):
        a0 = (2 * t) * GC2
        a1 = (2 * t + 1) * GC2
        a2 = (2 * t + 2) * GC2
        for b in range(GC2):
            drain_g(a0 + b, b, 0)
        sd0 = [d for b in range(GC2) for d in fire_s(a0 + b, b, 0)]
        for b in range(GC2):
            fire_g(a1 + b, b, 1)
        for d in sd0:
            d.wait()
        for b in range(GC2):
            fire_g(a2 + b, b, 0)
        for b in range(GC2):
            drain_g(a1 + b, b, 1)
        sd1 = [d for b in range(GC2) for d in fire_s(a1 + b, b, 1)]
        for d in sd1:
            d.wait()
        return carry

    lax.fori_loop(0, (NGC - 1) // 2, pair, 0)
    aL = (NGC - 1) * GC2
    for b in range(GC2):
        drain_g(aL + b, b, 0)
    sdL = [d for b in range(GC2) for d in fire_s(aL + b, b, 0)]
    for d in sdL:
        d.wait()
    plsc.subcore_barrier()

    pltpu.sync_copy(t0_sp.at[pl.ds(base_n, NPN)], t0c_v)
    pltpu.sync_copy(t1_sp.at[pl.ds(base_n, NPN)], t1c_v)
    pltpu.sync_copy(dinv_hbm.at[pl.ds(base_n, NPN)], dv_v)

    @pl.when(cid == 0)
    def _():
        pltpu.sync_copy(zzb0_hbm.at[pl.ds(base_n, NPN)], zb0_v)
        pltpu.sync_copy(zzb1_hbm.at[pl.ds(base_n, NPN)], zb1_v)

    @pl.when(cid != 0)
    def _():
        def zb(j, carry):
            zb0_v[pl.ds(j * L, L)] = jnp.zeros((L,), _f32)
            zb1_v[pl.ds(j * L, L)] = jnp.zeros((L,), _f32)
            return carry

        lax.fori_loop(0, NPN // L, zb, 0)

    def nstep(j, carry):
        o = j * L
        dv = dv_v[pl.ds(o, L)]
        u0_v[pl.ds(o, L)] = dv * (t0c_v[pl.ds(o, L)] + zb0_v[pl.ds(o, L)])
        u1_v[pl.ds(o, L)] = dv * (t1c_v[pl.ds(o, L)] + zb1_v[pl.ds(o, L)])
        return carry

    lax.fori_loop(0, NPN // L, nstep, 0)
    for c5 in range(NPN // 128):
        pltpu.sync_copy(bidx_hbm.at[pl.ds(base_n + c5 * 128, 128)],
                        bibuf_v)
        pltpu.sync_copy(u0_v.at[pl.ds(c5 * 128, 128)],
                        pool0_sp.at[bibuf_v], add=True)
        pltpu.sync_copy(u1_v.at[pl.ds(c5 * 128, 128)],
                        pool1_sp.at[bibuf_v], add=True)

    plsc.subcore_barrier()

    @pl.when(sid == 0)
    def _():
        pltpu.sync_copy(pool0_sp, out0_hbm.at[cid])
        pltpu.sync_copy(pool1_sp, out1_hbm.at[cid])


def _tc0_body(x_ref, w_ref, dinv_ref, hh_ref):
    h = jnp.dot(x_ref[...], w_ref[...], preferred_element_type=_f32)
    hh_ref[...] = h * dinv_ref[...][:, None]


def _tc0_call(x_pad, W1, dinv):
    return pl.pallas_call(
        _tc0_body,
        grid=(GRID,),
        in_specs=[
            pl.BlockSpec((R, F), lambda i: (i, 0)),
            pl.BlockSpec((F, H), lambda i: (0, 0)),
            pl.BlockSpec((R,), lambda i: (i,)),
        ],
        out_specs=pl.BlockSpec((R, H), lambda i: (i, 0)),
        out_shape=jax.ShapeDtypeStruct((NPAD, H), _f32),
    )(x_pad, W1, dinv)


def _tc2_body(accp_ref, hh_ref, dinv_ref, b1_ref, w2_ref, b2_ref,
              zz0_ref, zz1_ref, zzb0_ref, zzb1_ref):
    i = pl.program_id(0)
    dinv = dinv_ref[...]
    a = accp_ref[0] + accp_ref[1] + hh_ref[...]
    y = jnp.maximum(a * dinv[:, None] + b1_ref[...][None, :], 0.0)
    z = jnp.dot(y, w2_ref[...], preferred_element_type=_f32)
    zz = z * dinv[:, None]
    rows = i * R + lax.broadcasted_iota(jnp.int32, (R,), 0)
    valid = (rows < N).astype(_f32)
    sdeg = 1.0 / dinv
    zz0_ref[...] = zz[:, 0] * valid
    zz1_ref[...] = zz[:, 1] * valid
    zzb0_ref[...] = (zz[:, 0] + b2_ref[0] * sdeg) * valid
    zzb1_ref[...] = (zz[:, 1] + b2_ref[1] * sdeg) * valid


def _tc2_call(accp, hh, dinv, b1, W2, b2):
    vec = jax.ShapeDtypeStruct((NPAD,), _f32)
    return pl.pallas_call(
        _tc2_body,
        grid=(GRID,),
        in_specs=[
            pl.BlockSpec((NC, R, H), lambda i: (0, i, 0)),
            pl.BlockSpec((R, H), lambda i: (i, 0)),
            pl.BlockSpec((R,), lambda i: (i,)),
            pl.BlockSpec((H,), lambda i: (0,)),
            pl.BlockSpec((H, C), lambda i: (0, 0)),
            pl.BlockSpec((C,), lambda i: (0,)),
        ],
        out_specs=[pl.BlockSpec((R,), lambda i: (i,))] * 4,
        out_shape=[vec] * 4,
    )(accp, hh, dinv, b1, W2, b2)


def kernel(x, edge_index, batch, W1, b1, W2, b2):
    src = edge_index[0]
    dst = edge_index[1]
    x_pad = jnp.pad(x, ((0, NPAD - N), (0, 0)))
    batch_pad = jnp.pad(batch, (0, NPAD - N))
    dinv = _deg_kernel(dst)
    hh = _tc0_call(x_pad, W1, dinv)
    accp = _acc_kernel(hh, src, dst)
    zz0, zz1, zzb0, zzb1 = _tc2_call(accp, hh, dinv, b1, W2, b2)
    out0p, out1p = _pool_kernel(src, dst, zz0, zz1, zzb0, zzb1,
                                dinv, batch_pad)
    return jnp.stack([out0p.sum(axis=0), out1p.sum(axis=0)], axis=1)


# speedup vs baseline: 1.0346x; 1.0346x over previous
import functools

import jax
import jax.numpy as jnp
from jax import lax
from jax.experimental import pallas as pl
from jax.experimental.pallas import tpu as pltpu
from jax.experimental.pallas import tpu_sc as plsc

N = 10000
F = 128
H = 128
C = 2
E = 320000
G = 64

NC = 2
NS = 16
L = 16
NW = NC * NS
NPN = 640
NPAD = NS * NPN
K = 80
EPW = E // NW
EPT = E // NS
CHB = EPW // K
CHC = EPT // K
GB = 5
NGB = CHB // GB
GA = 5
GC = 10
GC2 = GC // 2
NGC = CHC // GC2
R = 1024
GRID = NPAD // R

_mesh = plsc.VectorSubcoreMesh(
    core_axis_name="c", subcore_axis_name="s", num_cores=NC, num_subcores=NS)

_f32 = jnp.float32


def _zero_vec(ref, n):
    for j in range(n // L):
        ref[pl.ds(j * L, L)] = jnp.zeros((L,), _f32)


CA = EPW // K
GA2 = 5
NGA = CA // GA2


@functools.partial(
    pl.kernel,
    out_type=jax.ShapeDtypeStruct((NC, NPAD), _f32),
    mesh=_mesh,
    scratch_types=[pltpu.VMEM((K,), jnp.int32)] * (2 * GA2) + [
        pltpu.VMEM((K,), _f32),
        pltpu.VMEM((K,), _f32),
        pltpu.VMEM_SHARED((NPAD,), _f32),
        pltpu.SemaphoreType.DMA,
        pltpu.SemaphoreType.DMA,
        pltpu.SemaphoreType.DMA,
        pltpu.SemaphoreType.DMA,
    ],
)
def _deg_kernel(dst_hbm, degp_hbm, *rest):
    ibufs = rest[:2 * GA2]
    (ones_v, zbuf_v, deg_sp, isem0, isem1, ssem0, ssem1) = rest[2 * GA2:]
    bank_i = (ibufs[:GA2], ibufs[GA2:])
    isems = (isem0, isem1)
    ssems = (ssem0, ssem1)
    cid = lax.axis_index("c")
    sid = lax.axis_index("s")
    wid = sid * NC + cid
    for j in range(K // L):
        ones_v[pl.ds(j * L, L)] = jnp.full((L,), 1.0, _f32)
    _zero_vec(zbuf_v, K)
    base_n = sid * NPN

    def zr(j, carry):
        pltpu.sync_copy(zbuf_v, deg_sp.at[pl.ds(base_n + j * K, K)])
        return carry

    lax.fori_loop(0, NPN // K, zr, 0)
    plsc.subcore_barrier()
    base_e = wid * EPW

    def fire_i(i, b, bk):
        pltpu.async_copy(dst_hbm.at[pl.ds(base_e + i * K, K)],
                         bank_i[bk][b], isems[bk])

    def drain_i(i, b, bk):
        pltpu.make_async_copy(dst_hbm.at[pl.ds(base_e + i * K, K)],
                              bank_i[bk][b], isems[bk]).wait()

    def fire_s(i, b, bk):
        return pltpu.async_copy(ones_v, deg_sp.at[bank_i[bk][b]],
                                ssems[bk], add=True)

    for b in range(GA2):
        fire_i(b, b, 0)

    def pair(t, carry):
        a0 = (2 * t) * GA2
        a1 = (2 * t + 1) * GA2
        a2 = (2 * t + 2) * GA2
        for b in range(GA2):
            drain_i(a0 + b, b, 0)
        sd0 = [fire_s(a0 + b, b, 0) for b in range(GA2)]
        for b in range(GA2):
            fire_i(a1 + b, b, 1)
        for d in sd0:
            d.wait()
        for b in range(GA2):
            fire_i(a2 + b, b, 0)
        for b in range(GA2):
            drain_i(a1 + b, b, 1)
        sd1 = [fire_s(a1 + b, b, 1) for b in range(GA2)]
        for d in sd1:
            d.wait()
        return carry

    lax.fori_loop(0, (NGA - 1) // 2, pair, 0)
    aL = (NGA - 1) * GA2
    for b in range(GA2):
        drain_i(aL + b, b, 0)
    sdL = [fire_s(aL + b, b, 0) for b in range(GA2)]
    for d in sdL:
        d.wait()
    plsc.subcore_barrier()
    pltpu.sync_copy(deg_sp.at[pl.ds(base_n, NPN)],
                    degp_hbm.at[cid, pl.ds(base_n, NPN)])


KB = 40
CB2 = EPW // KB
GB = 3
NGB = 83


@functools.partial(
    pl.kernel,
    out_type=jax.ShapeDtypeStruct((NC, NPAD, H), _f32),
    mesh=_mesh,
    scratch_types=[
        pltpu.VMEM((EPW,), jnp.int32),
    ] + [pltpu.VMEM((KB, H), _f32)] * (2 * GB)
      + [pltpu.VMEM((KB,), jnp.int32)] * (2 * GB) + [
        pltpu.VMEM_SHARED((NPAD, H), _f32),
        pltpu.SemaphoreType.DMA,
        pltpu.SemaphoreType.DMA,
        pltpu.SemaphoreType.DMA,
        pltpu.SemaphoreType.DMA,
    ],
)
def _acc_kernel(hh_hbm, src_hbm, dst_hbm, accp_hbm, sidx_all, *rest):
    rows = rest[:2 * GB]
    ibufs = rest[2 * GB:4 * GB]
    acc_sp, gsem0, gsem1, ssem0, ssem1 = rest[4 * GB:]
    bank_r = (rows[:GB], rows[GB:])
    bank_i = (ibufs[:GB], ibufs[GB:])
    gsems = (gsem0, gsem1)
    ssems = (ssem0, ssem1)
    cid = lax.axis_index("c")
    sid = lax.axis_index("s")
    wid = sid * NC + cid
    zrow = rows[0]

    def zf(i, carry):
        r = i // (H // L)
        c8 = (i % (H // L)) * L
        zrow[r, pl.ds(c8, L)] = jnp.zeros((L,), _f32)
        return carry

    lax.fori_loop(0, KB * (H // L), zf, 0)
    base_n = sid * NPN

    def zr(j, carry):
        pltpu.sync_copy(zrow, acc_sp.at[pl.ds(base_n + j * KB, KB)])
        return carry

    lax.fori_loop(0, NPN // KB, zr, 0)
    base_e = wid * EPW
    pltpu.sync_copy(src_hbm.at[pl.ds(base_e, EPW)], sidx_all)
    plsc.subcore_barrier()

    def g_src(i):
        return hh_hbm.at[sidx_all.at[pl.ds(i * KB, KB)]]

    def i_src(i):
        return dst_hbm.at[pl.ds(base_e + i * KB, KB)]

    def fire_g(i, b, bk):
        pltpu.async_copy(g_src(i), bank_r[bk][b], gsems[bk])
        pltpu.async_copy(i_src(i), bank_i[bk][b], gsems[bk])

    def drain_g(i, b, bk):
        pltpu.make_async_copy(g_src(i), bank_r[bk][b], gsems[bk]).wait()
        pltpu.make_async_copy(i_src(i), bank_i[bk][b], gsems[bk]).wait()

    def fire_s(i, b, bk):
        return pltpu.async_copy(
            bank_r[bk][b], acc_sp.at[bank_i[bk][b]], ssems[bk], add=True)

    for b in range(GB):
        fire_g(b, b, 0)

    def pair(t, carry):
        a0 = (2 * t) * GB
        a1 = (2 * t + 1) * GB
        a2 = (2 * t + 2) * GB
        for b in range(GB):
            drain_g(a0 + b, b, 0)
        sd0 = [fire_s(a0 + b, b, 0) for b in range(GB)]
        for b in range(GB):
            fire_g(a1 + b, b, 1)
        for d in sd0:
            d.wait()
        for b in range(GB):
            fire_g(a2 + b, b, 0)
        for b in range(GB):
            drain_g(a1 + b, b, 1)
        sd1 = [fire_s(a1 + b, b, 1) for b in range(GB)]
        for d in sd1:
            d.wait()
        return carry

    lax.fori_loop(0, (NGB - 1) // 2, pair, 0)
    aL = (NGB - 1) * GB
    for b in range(GB):
        drain_g(aL + b, b, 0)
    sdL = [fire_s(aL + b, b, 0) for b in range(GB)]
    for d in sdL:
        d.wait()
    for i in range(NGB * GB, CB2):
        fire_g(i, 0, 0)
        drain_g(i, 0, 0)
        fire_s(i, 0, 0).wait()
    plsc.subcore_barrier()
    pltpu.sync_copy(acc_sp.at[pl.ds(base_n, NPN)],
                    accp_hbm.at[cid, pl.ds(base_n, NPN)])


CC = EPW // K
GC2 = 5
NGC = CC // GC2


@functools.partial(
    pl.kernel,
    out_type=(jax.ShapeDtypeStruct((NC, G), _f32),
              jax.ShapeDtypeStruct((NC, G), _f32)),
    mesh=_mesh,
    scratch_types=[
        pltpu.VMEM((EPW,), jnp.int32),
    ] + [pltpu.VMEM((K,), _f32)] * (4 * GC2)
      + [pltpu.VMEM((K,), jnp.int32)] * (2 * GC2) + [
        pltpu.VMEM((K,), _f32),
        pltpu.VMEM((NPN,), _f32),
        pltpu.VMEM((NPN,), _f32),
        pltpu.VMEM((NPN,), _f32),
        pltpu.VMEM((NPN,), _f32),
        pltpu.VMEM((NPN,), _f32),
        pltpu.VMEM((NPN,), _f32),
        pltpu.VMEM((NPN,), _f32),
        pltpu.VMEM((128,), jnp.int32),
        pltpu.VMEM_SHARED((NPAD,), _f32),
        pltpu.VMEM_SHARED((NPAD,), _f32),
        pltpu.VMEM_SHARED((G,), _f32),
        pltpu.VMEM_SHARED((G,), _f32),
        pltpu.SemaphoreType.DMA,
        pltpu.SemaphoreType.DMA,
        pltpu.SemaphoreType.DMA,
        pltpu.SemaphoreType.DMA,
    ],
)
def _pool_kernel(src_hbm, dst_hbm, zz0_hbm, zz1_hbm, zzb0_hbm, zzb1_hbm,
                 dinv_hbm, bidx_hbm, out0_hbm, out1_hbm,
                 sidx_all, *rest):
    vbufs = rest[:4 * GC2]
    ibufs = rest[4 * GC2:6 * GC2]
    (zbuf_v, t0c_v, t1c_v, zb0_v, zb1_v, dv_v, u0_v, u1_v, bibuf_v,
     t0_sp, t1_sp, pool0_sp, pool1_sp,
     gsem0, gsem1, ssem0, ssem1) = rest[6 * GC2:]
    bank_0 = (vbufs[:GC2], vbufs[GC2:2 * GC2])
    bank_1 = (vbufs[2 * GC2:3 * GC2], vbufs[3 * GC2:])
    bank_i = (ibufs[:GC2], ibufs[GC2:])
    gsems = (gsem0, gsem1)
    ssems = (ssem0, ssem1)
    cid = lax.axis_index("c")
    sid = lax.axis_index("s")
    wid = sid * NC + cid
    _zero_vec(zbuf_v, K)
    base_n = sid * NPN

    def zr(j, carry):
        pltpu.sync_copy(zbuf_v, t0_sp.at[pl.ds(base_n + j * K, K)])
        pltpu.sync_copy(zbuf_v, t1_sp.at[pl.ds(base_n + j * K, K)])
        return carry

    lax.fori_loop(0, NPN // K, zr, 0)
    base_e = wid * EPW
    pltpu.sync_copy(src_hbm.at[pl.ds(base_e, EPW)], sidx_all)

    @pl.when(sid == 0)
    def _():
        pltpu.sync_copy(zbuf_v.at[pl.ds(0, G)], pool0_sp)
        pltpu.sync_copy(zbuf_v.at[pl.ds(0, G)], pool1_sp)

    plsc.subcore_barrier()

    def fire_g(i, b, bk):
        pltpu.async_copy(dst_hbm.at[pl.ds(base_e + i * K, K)],
                         bank_i[bk][b], gsems[bk])
        pltpu.async_copy(zz0_hbm.at[sidx_all.at[pl.ds(i * K, K)]],
                         bank_0[bk][b], gsems[bk])
        pltpu.async_copy(zz1_hbm.at[sidx_all.at[pl.ds(i * K, K)]],
                         bank_1[bk][b], gsems[bk])

    def drain_g(i, b, bk):
        pltpu.make_async_copy(dst_hbm.at[pl.ds(base_e + i * K, K)],
                              bank_i[bk][b], gsems[bk]).wait()
        pltpu.make_async_copy(zz0_hbm.at[sidx_all.at[pl.ds(i * K, K)]],
                              bank_0[bk][b], gsems[bk]).wait()
        pltpu.make_async_copy(zz1_hbm.at[sidx_all.at[pl.ds(i * K, K)]],
                              bank_1[bk][b], gsems[bk]).wait()

    def fire_s(i, b, bk):
        return [pltpu.async_copy(bank_0[bk][b], t0_sp.at[bank_i[bk][b]],
                                 ssems[bk], add=True),
                pltpu.async_copy(bank_1[bk][b], t1_sp.at[bank_i[bk][b]],
                                 ssems[bk], add=True)]

    for b in range(GC2):
        fire_g(b, b, 0)

    def pair(t, carry):
        a0 = (2 * t) * GC2
        a1 = (2 * t + 1) * GC2
        a2 = (2 * t + 2) * GC2
        for b in range(GC2):
            drain_g(a0 + b, b, 0)
        sd0 = [d for b in range(GC2) for d in fire_s(a0 + b, b, 0)]
        for b in range(GC2):
            fire_g(a1 + b, b, 1)
        for d in sd0:
            d.wait()
        for b in range(GC2):
            fire_g(a2 + b, b, 0)
        for b in range(GC2):
            drain_g(a1 + b, b, 1)
        sd1 = [d for b in range(GC2) for d in fire_s(a1 + b, b, 1)]
        for d in sd1:
            d.wait()
        return carry

    lax.fori_loop(0, (NGC - 1) // 2, pair, 0)
    aL = (NGC - 1) * GC2
    for b in range(GC2):
        drain_g(aL + b, b, 0)
    sdL = [d for b in range(GC2) for d in fire_s(aL + b, b, 0)]
    for d in sdL:
        d.wait()
    plsc.subcore_barrier()

    pltpu.sync_copy(t0_sp.at[pl.ds(base_n, NPN)], t0c_v)
    pltpu.sync_copy(t1_sp.at[pl.ds(base_n, NPN)], t1c_v)
    pltpu.sync_copy(dinv_hbm.at[pl.ds(base_n, NPN)], dv_v)

    @pl.when(cid == 0)
    def _():
        pltpu.sync_copy(zzb0_hbm.at[pl.ds(base_n, NPN)], zb0_v)
        pltpu.sync_copy(zzb1_hbm.at[pl.ds(base_n, NPN)], zb1_v)

    @pl.when(cid != 0)
    def _():
        def zb(j, carry):
            zb0_v[pl.ds(j * L, L)] = jnp.zeros((L,), _f32)
            zb1_v[pl.ds(j * L, L)] = jnp.zeros((L,), _f32)
            return carry

        lax.fori_loop(0, NPN // L, zb, 0)

    def nstep(j, carry):
        o = j * L
        dv = dv_v[pl.ds(o, L)]
        u0_v[pl.ds(o, L)] = dv * (t0c_v[pl.ds(o, L)] + zb0_v[pl.ds(o, L)])
        u1_v[pl.ds(o, L)] = dv * (t1c_v[pl.ds(o, L)] + zb1_v[pl.ds(o, L)])
        return carry

    lax.fori_loop(0, NPN // L, nstep, 0)
    for c5 in range(NPN // 128):
        pltpu.sync_copy(bidx_hbm.at[pl.ds(base_n + c5 * 128, 128)],
                        bibuf_v)
        pltpu.sync_copy(u0_v.at[pl.ds(c5 * 128, 128)],
                        pool0_sp.at[bibuf_v], add=True)
        pltpu.sync_copy(u1_v.at[pl.ds(c5 * 128, 128)],
                        pool1_sp.at[bibuf_v], add=True)

    plsc.subcore_barrier()

    @pl.when(sid == 0)
    def _():
        pltpu.sync_copy(pool0_sp, out0_hbm.at[cid])
        pltpu.sync_copy(pool1_sp, out1_hbm.at[cid])


def _tc0_body(x_ref, w_ref, degp_ref, hh_ref, dinv_ref):
    deg = degp_ref[0, :] + degp_ref[1, :] + 1.0
    dinv = lax.rsqrt(deg)
    h = jnp.dot(x_ref[...], w_ref[...], preferred_element_type=_f32)
    hh_ref[...] = h * dinv[:, None]
    dinv_ref[...] = dinv


def _tc0_call(x_pad, W1, degp):
    return pl.pallas_call(
        _tc0_body,
        grid=(GRID,),
        in_specs=[
            pl.BlockSpec((R, F), lambda i: (i, 0)),
            pl.BlockSpec((F, H), lambda i: (0, 0)),
            pl.BlockSpec((NC, R), lambda i: (0, i)),
        ],
        out_specs=[
            pl.BlockSpec((R, H), lambda i: (i, 0)),
            pl.BlockSpec((R,), lambda i: (i,)),
        ],
        out_shape=[
            jax.ShapeDtypeStruct((NPAD, H), _f32),
            jax.ShapeDtypeStruct((NPAD,), _f32),
        ],
    )(x_pad, W1, degp)


def _tc2_body(accp_ref, hh_ref, dinv_ref, b1_ref, w2_ref, b2_ref,
              zz0_ref, zz1_ref, zzb0_ref, zzb1_ref):
    i = pl.program_id(0)
    dinv = dinv_ref[...]
    a = accp_ref[0] + accp_ref[1] + hh_ref[...]
    y = jnp.maximum(a * dinv[:, None] + b1_ref[...][None, :], 0.0)
    z = jnp.dot(y, w2_ref[...], preferred_element_type=_f32)
    zz = z * dinv[:, None]
    rows = i * R + lax.broadcasted_iota(jnp.int32, (R,), 0)
    valid = (rows < N).astype(_f32)
    sdeg = 1.0 / dinv
    zz0_ref[...] = zz[:, 0] * valid
    zz1_ref[...] = zz[:, 1] * valid
    zzb0_ref[...] = (zz[:, 0] + b2_ref[0] * sdeg) * valid
    zzb1_ref[...] = (zz[:, 1] + b2_ref[1] * sdeg) * valid


def _tc2_call(accp, hh, dinv, b1, W2, b2):
    vec = jax.ShapeDtypeStruct((NPAD,), _f32)
    return pl.pallas_call(
        _tc2_body,
        grid=(GRID,),
        in_specs=[
            pl.BlockSpec((NC, R, H), lambda i: (0, i, 0)),
            pl.BlockSpec((R, H), lambda i: (i, 0)),
            pl.BlockSpec((R,), lambda i: (i,)),
            pl.BlockSpec((H,), lambda i: (0,)),
            pl.BlockSpec((H, C), lambda i: (0, 0)),
            pl.BlockSpec((C,), lambda i: (0,)),
        ],
        out_specs=[pl.BlockSpec((R,), lambda i: (i,))] * 4,
        out_shape=[vec] * 4,
    )(accp, hh, dinv, b1, W2, b2)


def kernel(x, edge_index, batch, W1, b1, W2, b2):
    src = edge_index[0]
    dst = edge_index[1]
    x_pad = jnp.pad(x, ((0, NPAD - N), (0, 0)))
    batch_pad = jnp.pad(batch, (0, NPAD - N))
    degp = _deg_kernel(dst)
    hh, dinv = _tc0_call(x_pad, W1, degp)
    accp = _acc_kernel(hh, src, dst)
    zz0, zz1, zzb0, zzb1 = _tc2_call(accp, hh, dinv, b1, W2, b2)
    out0p, out1p = _pool_kernel(src, dst, zz0, zz1, zzb0, zzb1,
                                dinv, batch_pad)
    return jnp.stack([out0p.sum(axis=0), out1p.sum(axis=0)], axis=1)
